# Initial kernel scaffold; baseline (speedup 1.0000x reference)
import jax
import jax.numpy as jnp
from jax.experimental import pallas as pl


def kernel(x):
    raise NotImplementedError("write your pallas kernel here")


# speedup vs baseline: 8.9326x; 8.9326x over previous
import jax
import jax.numpy as jnp
import numpy as np
from jax.experimental import pallas as pl
from jax.experimental.pallas import tpu as pltpu

_K = 128
_N = 32768
_ROWS = 128
_BLOCK_ROWS = 8
_INT_MIN = np.int32(-2147483648)


def _topk_mean_body(x_ref, o_ref):
    x = x_ref[...]
    b = pltpu.bitcast(x, jnp.int32)
    key = jnp.where(b >= 0, b, _INT_MIN - b)

    t_u = jnp.zeros((_BLOCK_ROWS, 1), jnp.int32)
    for i in range(31, -1, -1):
        cand_u = t_u | np.int32(1 << i) if i < 31 else t_u | _INT_MIN
        cand_s = cand_u ^ _INT_MIN
        cnt = jnp.sum((key >= cand_s).astype(jnp.int32), axis=1, keepdims=True)
        t_u = jnp.where(cnt >= _K, cand_u, t_u)

    t_s = t_u ^ _INT_MIN
    above = key > t_s
    s_above = jnp.sum(jnp.where(above, x, 0.0), axis=1, keepdims=True)
    c_above = jnp.sum(above.astype(jnp.int32), axis=1, keepdims=True)
    tb = jnp.where(t_s >= 0, t_s, _INT_MIN - t_s)
    t_val = pltpu.bitcast(tb, jnp.float32)
    total = s_above + (_K - c_above).astype(jnp.float32) * t_val
    o_ref[...] = total * (1.0 / _K)


def kernel(x):
    grid = _ROWS // _BLOCK_ROWS
    return pl.pallas_call(
        _topk_mean_body,
        grid=(grid,),
        in_specs=[pl.BlockSpec((_BLOCK_ROWS, _N), lambda i: (i, 0))],
        out_specs=pl.BlockSpec((_BLOCK_ROWS, 1), lambda i: (i, 0)),
        out_shape=jax.ShapeDtypeStruct((_ROWS, 1), jnp.float32),
    )(x).reshape(_ROWS)
